# baseline (speedup 1.0000x reference)
import functools

import jax
import jax.numpy as jnp
from jax import lax
from jax.experimental import pallas as pl
from jax.experimental.pallas import tpu as pltpu
from jax.experimental.pallas import tpu_sc as plsc

BATCH = 16384
EMBED_DIM = 16
NUM_CORES = 2
NUM_SUBCORES = 16
NUM_WORKERS = NUM_CORES * NUM_SUBCORES
B_PER_W = BATCH // NUM_WORKERS
LANES = 16


def _fm_body(uid_hbm, tid_hbm, utab_hbm, itab_hbm, wv_hbm, bv_hbm, out_hbm,
             idx_u_v, idx_t_v, rows_u_v, rows_t_v, out_v, wv_v, bv_v,
             sem_u, sem_t):
    wid = lax.axis_index("s") * NUM_CORES + lax.axis_index("c")
    base = wid * B_PER_W

    pltpu.sync_copy(uid_hbm.at[pl.ds(base, B_PER_W)], idx_u_v)
    pltpu.sync_copy(tid_hbm.at[pl.ds(base, B_PER_W)], idx_t_v)
    cu = pltpu.async_copy(utab_hbm.at[idx_u_v], rows_u_v, sem_u)
    ct = pltpu.async_copy(itab_hbm.at[idx_t_v], rows_t_v, sem_t)
    pltpu.sync_copy(wv_hbm, wv_v)
    pltpu.sync_copy(bv_hbm, bv_v)
    cu.wait()
    ct.wait()

    wv = wv_v[...]
    bv = bv_v[...]
    lanes = lax.iota(jnp.int32, LANES)

    def chunk_body(j, carry):
        row_idx = j * LANES + lanes
        acc = jnp.zeros((LANES,), jnp.float32)
        for d in range(EMBED_DIM):
            col = jnp.full((LANES,), d, jnp.int32)
            cu = plsc.load_gather(rows_u_v, [row_idx, col])
            ct = plsc.load_gather(rows_t_v, [row_idx, col])
            acc = acc + cu * ct
        z = acc * wv + bv
        y = 1.0 / (1.0 + jnp.exp(-z))
        out_v[pl.ds(j * LANES, LANES)] = y
        return carry

    lax.fori_loop(0, B_PER_W // LANES, chunk_body, 0)
    pltpu.sync_copy(out_v, out_hbm.at[pl.ds(base, B_PER_W)])


@jax.jit
def _fm_sc(f_uid, f_tid, user_table, item_table, wv, bv):
    mesh = plsc.VectorSubcoreMesh(core_axis_name="c", subcore_axis_name="s")
    return pl.kernel(
        _fm_body,
        out_type=jax.ShapeDtypeStruct((BATCH,), jnp.float32),
        mesh=mesh,
        compiler_params=pltpu.CompilerParams(
            needs_layout_passes=False, use_tc_tiling_on_sc=False),
        scratch_types=[
            pltpu.VMEM((B_PER_W,), jnp.int32),
            pltpu.VMEM((B_PER_W,), jnp.int32),
            pltpu.VMEM((B_PER_W, EMBED_DIM), jnp.float32),
            pltpu.VMEM((B_PER_W, EMBED_DIM), jnp.float32),
            pltpu.VMEM((B_PER_W,), jnp.float32),
            pltpu.VMEM((LANES,), jnp.float32),
            pltpu.VMEM((LANES,), jnp.float32),
            pltpu.SemaphoreType.DMA,
            pltpu.SemaphoreType.DMA,
        ],
    )(f_uid, f_tid, user_table, item_table, wv, bv)


def kernel(f_uid, f_tid, user_table, item_table, W, b):
    wv = jnp.broadcast_to(W.reshape(()), (LANES,))
    bv = jnp.broadcast_to(b.reshape(()), (LANES,))
    y = _fm_sc(f_uid.astype(jnp.int32), f_tid.astype(jnp.int32),
               user_table, item_table, wv, bv)
    return y.reshape(BATCH, 1)
